# baseline (speedup 1.0000x reference)
import jax
import jax.numpy as jnp
import numpy as np
from jax.experimental import pallas as pl

B = 16
H = 512
Z = 128
A = 8
CAND = 256
ITERS = 2
T = 12
TOPK = 32
MAXA = 1.0
MINA = -1.0
D = 512

_G = 16


def _draw_noise():
    key = jax.random.key(42)
    out = []
    for _ in range(ITERS):
        key, sub = jax.random.split(key)
        n = jax.random.normal(sub, (T, B, CAND, A), dtype=jnp.float32)
        out.append(n.reshape(T, B * CAND, A))
    return out


_NOISES = [np.asarray(x) for x in jax.jit(_draw_noise)()]
_NOISES_PACKED = [n.reshape(T, CAND, 128) for n in _NOISES]

_BF = jnp.bfloat16


def _rollout_body(noise_ref, mean_ref, std_ref, belief_ref, state_ref,
                  wzb_ref, ws_ref, wa_ref, w1_ref, w2_ref, ret_ref):
    b = jnp.broadcast_to(belief_ref[0].astype(_BF), (CAND, H))
    s = jnp.broadcast_to(state_ref[0].astype(_BF), (CAND, Z))
    mean = mean_ref[0]
    std = std_ref[0]
    one = mean[0, 0] * 0.0 + 1.0
    y = jnp.dot(b, wzb_ref[:], preferred_element_type=jnp.float32)
    xfs = []
    for t in range(T):
        a_t = jnp.clip(mean[t][None, :] + std[t][None, :] * noise_ref[t],
                       MINA, MAXA).astype(_BF)
        d1 = y[:, Z:]
        d2 = jnp.dot(s, ws_ref[:], preferred_element_type=jnp.float32)
        d3 = jnp.dot(a_t, wa_ref[:], preferred_element_type=jnp.float32)
        b = jnp.tanh((d1 * one + d2 * one) + d3).astype(_BF)
        y = jnp.dot(b, wzb_ref[:], preferred_element_type=jnp.float32)
        s = jnp.tanh(y[:, :Z]).astype(_BF)
        xfs.append(jnp.concatenate([b, s], axis=1))
    xcat = jnp.concatenate(xfs, axis=0)
    hcat = jnp.tanh(jnp.dot(xcat, w1_ref[:],
                            preferred_element_type=jnp.float32)).astype(_BF)
    r_all = jnp.dot(hcat, w2_ref[:], preferred_element_type=jnp.float32)
    ret_ref[0, 0, :] = jnp.sum(r_all.reshape(T, CAND), axis=0)


def _topk_mask(r):
    iota = jax.lax.broadcasted_iota(jnp.int32, (B, CAND), 1)
    w = jnp.zeros((B, CAND), jnp.float32)
    for _ in range(TOPK):
        m = jnp.max(r, axis=1, keepdims=True)
        is_max = r == m
        idx = jnp.min(jnp.where(is_max, iota, CAND), axis=1, keepdims=True)
        first = iota == idx
        w = jnp.where(first, 1.0, w)
        r = jnp.where(first, -jnp.inf, r)
    return w


def _packed(x16):
    return jnp.broadcast_to(x16[:, None, None, :], (B, _G, _G, A)).reshape(
        CAND, 128)


def _packed_mask(w):
    return jnp.broadcast_to(w.reshape(B, _G, _G)[:, :, :, None],
                            (B, _G, _G, A)).reshape(CAND, 128)


def _lane_groupsum(x):
    for sh in (64, 32, 16, 8):
        x = x + jnp.roll(x, -sh, axis=1)
    return x[:, :A]


def _moments_body(ret_ref, npk_ref, mean_ref, std_ref,
                  mean_out_ref, std_out_ref):
    w = _topk_mask(ret_ref[:, 0, :])
    wp = _packed_mask(w)
    inv_k = 1.0 / TOPK
    for t in range(T):
        mp = _packed(mean_ref[:, t, :])
        sp = _packed(std_ref[:, t, :])
        ap = jnp.clip(mp + sp * npk_ref[t], MINA, MAXA)
        mean_t = _lane_groupsum(ap * wp).reshape(B, _G, A).sum(axis=1) * inv_k
        d = (ap - _packed(mean_t)) * wp
        var_t = _lane_groupsum(d * d).reshape(B, _G, A).sum(axis=1) * inv_k
        mean_out_ref[:, t, :] = mean_t
        std_out_ref[:, t, :] = jnp.sqrt(var_t)


def _moments_final_body(ret_ref, npk_ref, mean_ref, std_ref, mean_out_ref):
    w = _topk_mask(ret_ref[:, 0, :])
    wp = _packed_mask(w)
    mp = _packed(mean_ref[:, 0, :])
    sp = _packed(std_ref[:, 0, :])
    ap = jnp.clip(mp + sp * npk_ref[0], MINA, MAXA)
    s1 = _lane_groupsum(ap * wp).reshape(B, _G, A).sum(axis=1)
    mean_out_ref[:, :] = s1 * (1.0 / TOPK)


def _rollout(noise, mean, std, belief3, state3, wzb, ws, wa, w1, w2col):
    return pl.pallas_call(
        _rollout_body,
        grid=(B,),
        in_specs=[
            pl.BlockSpec((T, CAND, A), lambda i: (0, i, 0)),
            pl.BlockSpec((1, T, A), lambda i: (i, 0, 0)),
            pl.BlockSpec((1, T, A), lambda i: (i, 0, 0)),
            pl.BlockSpec((1, 1, H), lambda i: (i, 0, 0)),
            pl.BlockSpec((1, 1, Z), lambda i: (i, 0, 0)),
            pl.BlockSpec((H, Z + H), lambda i: (0, 0)),
            pl.BlockSpec((Z, H), lambda i: (0, 0)),
            pl.BlockSpec((A, H), lambda i: (0, 0)),
            pl.BlockSpec((H + Z, D), lambda i: (0, 0)),
            pl.BlockSpec((D, 1), lambda i: (0, 0)),
        ],
        out_specs=pl.BlockSpec((1, 1, CAND), lambda i: (i, 0, 0)),
        out_shape=jax.ShapeDtypeStruct((B, 1, CAND), jnp.float32),
    )(noise, mean, std, belief3, state3, wzb, ws, wa, w1, w2col)


def _moments(returns, npk, mean, std):
    return pl.pallas_call(
        _moments_body,
        out_shape=[
            jax.ShapeDtypeStruct((B, T, A), jnp.float32),
            jax.ShapeDtypeStruct((B, T, A), jnp.float32),
        ],
    )(returns, npk, mean, std)


def _moments_final(returns, npk, mean, std):
    return pl.pallas_call(
        _moments_final_body,
        out_shape=jax.ShapeDtypeStruct((B, A), jnp.float32),
    )(returns, npk, mean, std)


def kernel(belief, state, Wb, Ws, Wa, Wz, W1, w2):
    wzb = jnp.concatenate([Wz, Wb], axis=1).astype(_BF)
    ws = Ws.astype(_BF)
    wa = Wa.astype(_BF)
    w1 = W1.astype(_BF)
    w2col = w2.reshape(D, 1).astype(_BF)
    belief3 = belief.reshape(B, 1, H)
    state3 = state.reshape(B, 1, Z)
    mean = jnp.zeros((B, T, A), jnp.float32)
    std = jnp.ones((B, T, A), jnp.float32)
    ret = _rollout(_NOISES[0], mean, std, belief3, state3,
                   wzb, ws, wa, w1, w2col)
    mean, std = _moments(ret, _NOISES_PACKED[0], mean, std)
    ret = _rollout(_NOISES[1], mean, std, belief3, state3,
                   wzb, ws, wa, w1, w2col)
    return _moments_final(ret, _NOISES_PACKED[1], mean, std)
